# baseline (speedup 1.0000x reference)
import functools

import jax
import jax.numpy as jnp
from jax.experimental import pallas as pl
from jax.experimental.pallas import tpu as pltpu

_TT = 256
_NCORES = 2


def _moe_unit_kernel(meta_ref, x_ref, gate_ref, up_ref, down_ref, out_ref, *, tt, half):
    c = pl.program_id(0)
    g = pl.program_id(1)

    @pl.when(g == 0)
    def _zero():
        out_ref[...] = jnp.zeros_like(out_ref)

    tile = meta_ref[c, 1, g]
    rs = meta_ref[c, 2, g]
    re = meta_ref[c, 3, g]

    @pl.when(rs < re)
    def _compute():
        x = x_ref[...]
        gw = gate_ref[0]
        uw = up_ref[0]
        dw = down_ref[0]
        dn = (((1,), (1,)), ((), ()))
        gg = jax.lax.dot_general(x, gw, dn, preferred_element_type=jnp.float32)
        uu = jax.lax.dot_general(x, uw, dn, preferred_element_type=jnp.float32)
        act = gg * jax.nn.sigmoid(gg) * uu
        y = jax.lax.dot_general(act, dw, dn, preferred_element_type=jnp.float32)
        rows = tile * tt + jax.lax.broadcasted_iota(jnp.int32, (tt, 1), 0)
        keep = (rows >= rs) & (rows < re)
        local = (tile - c * (half // tt)) * tt
        sl = pl.ds(local, tt)
        out_ref[sl, :] = jnp.where(keep, y, out_ref[sl, :])


def _build_units(seg_lo, seg_hi, tile0, n_tiles, tt, n_units):
    e = seg_lo.shape[0]
    lo = jnp.clip(seg_lo, tile0 * tt, (tile0 + n_tiles) * tt)
    hi = jnp.clip(seg_hi, tile0 * tt, (tile0 + n_tiles) * tt)
    nonempty = hi > lo
    first_tile = jnp.where(nonempty, lo // tt, 0)
    ntiles = jnp.where(nonempty, (hi - 1) // tt - first_tile + 1, 0)
    ii = jnp.arange(e, dtype=jnp.int32)
    cum = jnp.sum(jnp.where(ii[None, :] <= ii[:, None], ntiles[None, :], 0), axis=1)
    total = jnp.sum(ntiles)
    u = jnp.arange(n_units, dtype=jnp.int32)
    eu = jnp.sum((cum[None, :] <= u[:, None]).astype(jnp.int32), axis=1)
    euc = jnp.minimum(eu, e - 1)
    oh = ii[None, :] == euc[:, None]

    def sel(v):
        return jnp.sum(jnp.where(oh, v[None, :], 0), axis=1)

    prev = sel(cum) - sel(ntiles)
    tile_u = sel(first_tile) + (u - prev)
    rs_u = jnp.maximum(sel(lo), tile_u * tt)
    re_u = jnp.minimum(sel(hi), (tile_u + 1) * tt)
    valid = u < total
    last = jnp.maximum(total - 1, 0)
    ohl = (u == last) & (total > 0)
    e_pad = jnp.sum(jnp.where(ohl, euc, 0))
    t_pad = jnp.sum(jnp.where(ohl, tile_u, 0))
    return jnp.stack([
        jnp.where(valid, euc, e_pad),
        jnp.where(valid, tile_u, t_pad),
        jnp.where(valid, rs_u, 0),
        jnp.where(valid, re_u, 0),
    ])


@jax.jit
def kernel(hidden_states, experts_indices, start_indices, end_indices, gate_w, up_w, down_w):
    del experts_indices
    t_tokens, d = hidden_states.shape
    e, ff, _ = gate_w.shape
    tt = _TT
    half = t_tokens // _NCORES
    tiles_half = half // tt
    n_units = tiles_half + e

    s = start_indices.astype(jnp.int32)
    seg_lo = s
    seg_hi = jnp.minimum(
        end_indices.astype(jnp.int32),
        jnp.concatenate([s[1:], jnp.full((1,), t_tokens, jnp.int32)]),
    )
    meta = jnp.stack([
        _build_units(seg_lo, seg_hi, c * tiles_half, tiles_half, tt, n_units)
        for c in range(_NCORES)
    ])

    grid_spec = pltpu.PrefetchScalarGridSpec(
        num_scalar_prefetch=1,
        grid=(_NCORES, n_units),
        in_specs=[
            pl.BlockSpec((tt, d), lambda c, g, m: (m[c, 1, g], 0)),
            pl.BlockSpec((1, ff, d), lambda c, g, m: (m[c, 0, g], 0, 0)),
            pl.BlockSpec((1, ff, d), lambda c, g, m: (m[c, 0, g], 0, 0)),
            pl.BlockSpec((1, d, ff), lambda c, g, m: (m[c, 0, g], 0, 0)),
        ],
        out_specs=pl.BlockSpec((half, d), lambda c, g, m: (c, 0)),
    )
    return pl.pallas_call(
        functools.partial(_moe_unit_kernel, tt=tt, half=half),
        grid_spec=grid_spec,
        out_shape=jax.ShapeDtypeStruct((t_tokens, d), jnp.float32),
        compiler_params=pltpu.CompilerParams(
            dimension_semantics=("parallel", "arbitrary"),
        ),
    )(meta, hidden_states, gate_w, up_w, down_w)


# speedup vs baseline: 1.0200x; 1.0200x over previous
import functools

import jax
import jax.numpy as jnp
from jax.experimental import pallas as pl
from jax.experimental.pallas import tpu as pltpu

_TT = 256
_NCORES = 2


def _moe_unit_kernel(meta_ref, x_ref, gate_ref, up_ref, down_ref, out_ref, *, tt, half):
    c = pl.program_id(0)
    g = pl.program_id(1)

    @pl.when(g == 0)
    def _zero():
        out_ref[...] = jnp.zeros_like(out_ref)

    tile = meta_ref[c, 1, g]
    rs = meta_ref[c, 2, g]
    re = meta_ref[c, 3, g]

    @pl.when(rs < re)
    def _compute():
        x = x_ref[...].astype(jnp.bfloat16)
        gw = gate_ref[0].astype(jnp.bfloat16)
        uw = up_ref[0].astype(jnp.bfloat16)
        dw = down_ref[0].astype(jnp.bfloat16)
        dn = (((1,), (1,)), ((), ()))
        gg = jax.lax.dot_general(x, gw, dn, preferred_element_type=jnp.float32)
        uu = jax.lax.dot_general(x, uw, dn, preferred_element_type=jnp.float32)
        act = (gg * jax.nn.sigmoid(gg) * uu).astype(jnp.bfloat16)
        y = jax.lax.dot_general(act, dw, dn, preferred_element_type=jnp.float32)
        rows = tile * tt + jax.lax.broadcasted_iota(jnp.int32, (tt, 1), 0)
        keep = (rows >= rs) & (rows < re)
        local = (tile - c * (half // tt)) * tt
        sl = pl.ds(local, tt)
        out_ref[sl, :] = jnp.where(keep, y, out_ref[sl, :])


def _build_units(seg_lo, seg_hi, tile0, n_tiles, tt, n_units):
    e = seg_lo.shape[0]
    lo = jnp.clip(seg_lo, tile0 * tt, (tile0 + n_tiles) * tt)
    hi = jnp.clip(seg_hi, tile0 * tt, (tile0 + n_tiles) * tt)
    nonempty = hi > lo
    first_tile = jnp.where(nonempty, lo // tt, 0)
    ntiles = jnp.where(nonempty, (hi - 1) // tt - first_tile + 1, 0)
    ii = jnp.arange(e, dtype=jnp.int32)
    cum = jnp.sum(jnp.where(ii[None, :] <= ii[:, None], ntiles[None, :], 0), axis=1)
    total = jnp.sum(ntiles)
    u = jnp.arange(n_units, dtype=jnp.int32)
    eu = jnp.sum((cum[None, :] <= u[:, None]).astype(jnp.int32), axis=1)
    euc = jnp.minimum(eu, e - 1)
    oh = ii[None, :] == euc[:, None]

    def sel(v):
        return jnp.sum(jnp.where(oh, v[None, :], 0), axis=1)

    prev = sel(cum) - sel(ntiles)
    tile_u = sel(first_tile) + (u - prev)
    rs_u = jnp.maximum(sel(lo), tile_u * tt)
    re_u = jnp.minimum(sel(hi), (tile_u + 1) * tt)
    valid = u < total
    last = jnp.maximum(total - 1, 0)
    ohl = (u == last) & (total > 0)
    e_pad = jnp.sum(jnp.where(ohl, euc, 0))
    t_pad = jnp.sum(jnp.where(ohl, tile_u, 0))
    return jnp.stack([
        jnp.where(valid, euc, e_pad),
        jnp.where(valid, tile_u, t_pad),
        jnp.where(valid, rs_u, 0),
        jnp.where(valid, re_u, 0),
    ])


@jax.jit
def kernel(hidden_states, experts_indices, start_indices, end_indices, gate_w, up_w, down_w):
    del experts_indices
    t_tokens, d = hidden_states.shape
    e, ff, _ = gate_w.shape
    tt = _TT
    half = t_tokens // _NCORES
    tiles_half = half // tt
    n_units = tiles_half + e

    s = start_indices.astype(jnp.int32)
    seg_lo = s
    seg_hi = jnp.minimum(
        end_indices.astype(jnp.int32),
        jnp.concatenate([s[1:], jnp.full((1,), t_tokens, jnp.int32)]),
    )
    meta = jnp.stack([
        _build_units(seg_lo, seg_hi, c * tiles_half, tiles_half, tt, n_units)
        for c in range(_NCORES)
    ])

    grid_spec = pltpu.PrefetchScalarGridSpec(
        num_scalar_prefetch=1,
        grid=(_NCORES, n_units),
        in_specs=[
            pl.BlockSpec((tt, d), lambda c, g, m: (m[c, 1, g], 0)),
            pl.BlockSpec((1, ff, d), lambda c, g, m: (m[c, 0, g], 0, 0)),
            pl.BlockSpec((1, ff, d), lambda c, g, m: (m[c, 0, g], 0, 0)),
            pl.BlockSpec((1, d, ff), lambda c, g, m: (m[c, 0, g], 0, 0)),
        ],
        out_specs=pl.BlockSpec((half, d), lambda c, g, m: (c, 0)),
    )
    return pl.pallas_call(
        functools.partial(_moe_unit_kernel, tt=tt, half=half),
        grid_spec=grid_spec,
        out_shape=jax.ShapeDtypeStruct((t_tokens, d), jnp.float32),
        compiler_params=pltpu.CompilerParams(
            dimension_semantics=("parallel", "arbitrary"),
        ),
    )(meta, hidden_states, gate_w, up_w, down_w)


# speedup vs baseline: 1.1350x; 1.1127x over previous
import functools

import jax
import jax.numpy as jnp
from jax.experimental import pallas as pl
from jax.experimental.pallas import tpu as pltpu

_TT = 256


def _moe_unit_kernel(meta_ref, x_ref, gate_ref, up_ref, down_ref, out_ref, *, tt):
    g = pl.program_id(0)

    @pl.when(g == 0)
    def _zero():
        out_ref[...] = jnp.zeros_like(out_ref)

    tile = meta_ref[1, g]
    rs = meta_ref[2, g]
    re = meta_ref[3, g]

    @pl.when(rs < re)
    def _compute():
        x = x_ref[...].astype(jnp.bfloat16)
        gw = gate_ref[0].astype(jnp.bfloat16)
        uw = up_ref[0].astype(jnp.bfloat16)
        dw = down_ref[0].astype(jnp.bfloat16)
        dn = (((1,), (1,)), ((), ()))
        gg = jax.lax.dot_general(x, gw, dn, preferred_element_type=jnp.float32)
        uu = jax.lax.dot_general(x, uw, dn, preferred_element_type=jnp.float32)
        act = (gg * jax.nn.sigmoid(gg) * uu).astype(jnp.bfloat16)
        y = jax.lax.dot_general(act, dw, dn, preferred_element_type=jnp.float32)
        rows = tile * tt + jax.lax.broadcasted_iota(jnp.int32, (tt, 1), 0)
        keep = (rows >= rs) & (rows < re)
        sl = pl.ds(tile * tt, tt)
        out_ref[sl, :] = jnp.where(keep, y, out_ref[sl, :])


def _build_units(seg_lo, seg_hi, n_tiles, tt, n_units):
    e = seg_lo.shape[0]
    lo = jnp.clip(seg_lo, 0, n_tiles * tt)
    hi = jnp.clip(seg_hi, 0, n_tiles * tt)
    nonempty = hi > lo
    first_tile = jnp.where(nonempty, lo // tt, 0)
    ntiles = jnp.where(nonempty, (hi - 1) // tt - first_tile + 1, 0)
    ii = jnp.arange(e, dtype=jnp.int32)
    cum = jnp.sum(jnp.where(ii[None, :] <= ii[:, None], ntiles[None, :], 0), axis=1)
    total = jnp.sum(ntiles)
    u = jnp.arange(n_units, dtype=jnp.int32)
    eu = jnp.sum((cum[None, :] <= u[:, None]).astype(jnp.int32), axis=1)
    euc = jnp.minimum(eu, e - 1)
    oh = ii[None, :] == euc[:, None]

    def sel(v):
        return jnp.sum(jnp.where(oh, v[None, :], 0), axis=1)

    prev = sel(cum) - sel(ntiles)
    tile_u = sel(first_tile) + (u - prev)
    rs_u = jnp.maximum(sel(lo), tile_u * tt)
    re_u = jnp.minimum(sel(hi), (tile_u + 1) * tt)
    valid = u < total
    last = jnp.maximum(total - 1, 0)
    ohl = (u == last) & (total > 0)
    e_pad = jnp.sum(jnp.where(ohl, euc, 0))
    t_pad = jnp.sum(jnp.where(ohl, tile_u, 0))
    return jnp.stack([
        jnp.where(valid, euc, e_pad),
        jnp.where(valid, tile_u, t_pad),
        jnp.where(valid, rs_u, 0),
        jnp.where(valid, re_u, 0),
    ])


@jax.jit
def kernel(hidden_states, experts_indices, start_indices, end_indices, gate_w, up_w, down_w):
    del experts_indices
    t_tokens, d = hidden_states.shape
    e, ff, _ = gate_w.shape
    tt = _TT
    n_tiles = t_tokens // tt
    n_units = n_tiles + e

    s = start_indices.astype(jnp.int32)
    seg_lo = s
    seg_hi = jnp.minimum(
        end_indices.astype(jnp.int32),
        jnp.concatenate([s[1:], jnp.full((1,), t_tokens, jnp.int32)]),
    )
    meta = _build_units(seg_lo, seg_hi, n_tiles, tt, n_units)

    grid_spec = pltpu.PrefetchScalarGridSpec(
        num_scalar_prefetch=1,
        grid=(n_units,),
        in_specs=[
            pl.BlockSpec((tt, d), lambda g, m: (m[1, g], 0)),
            pl.BlockSpec((1, ff, d), lambda g, m: (m[0, g], 0, 0)),
            pl.BlockSpec((1, ff, d), lambda g, m: (m[0, g], 0, 0)),
            pl.BlockSpec((1, d, ff), lambda g, m: (m[0, g], 0, 0)),
        ],
        out_specs=pl.BlockSpec((t_tokens, d), lambda g, m: (0, 0)),
    )
    return pl.pallas_call(
        functools.partial(_moe_unit_kernel, tt=tt),
        grid_spec=grid_spec,
        out_shape=jax.ShapeDtypeStruct((t_tokens, d), jnp.float32),
        compiler_params=pltpu.CompilerParams(
            dimension_semantics=("arbitrary",),
        ),
    )(meta, hidden_states, gate_w, up_w, down_w)


# speedup vs baseline: 1.1394x; 1.0039x over previous
import functools

import jax
import jax.numpy as jnp
from jax.experimental import pallas as pl
from jax.experimental.pallas import tpu as pltpu

_TT = 256


def _moe_unit_kernel(meta_ref, x_ref, gate_ref, up_ref, down_ref, out_ref, *, tt):
    g = pl.program_id(0)

    @pl.when(g == 0)
    def _zero():
        out_ref[...] = jnp.zeros_like(out_ref)

    tile = meta_ref[1, g]
    rs = meta_ref[2, g]
    re = meta_ref[3, g]

    @pl.when(rs < re)
    def _compute():
        x = x_ref[...]
        gw = gate_ref[0]
        uw = up_ref[0]
        dw = down_ref[0]
        dn = (((1,), (1,)), ((), ()))
        gg = jax.lax.dot_general(x, gw, dn, preferred_element_type=jnp.float32)
        uu = jax.lax.dot_general(x, uw, dn, preferred_element_type=jnp.float32)
        act = gg * jax.nn.sigmoid(gg) * uu
        y = jax.lax.dot_general(act, dw, dn, preferred_element_type=jnp.float32)
        rows = tile * tt + jax.lax.broadcasted_iota(jnp.int32, (tt, 1), 0)
        keep = (rows >= rs) & (rows < re)
        sl = pl.ds(tile * tt, tt)
        out_ref[sl, :] = jnp.where(keep, y, out_ref[sl, :])


def _build_units(seg_lo, seg_hi, n_tiles, tt, n_units):
    e = seg_lo.shape[0]
    lo = jnp.clip(seg_lo, 0, n_tiles * tt)
    hi = jnp.clip(seg_hi, 0, n_tiles * tt)
    nonempty = hi > lo
    first_tile = jnp.where(nonempty, lo // tt, 0)
    ntiles = jnp.where(nonempty, (hi - 1) // tt - first_tile + 1, 0)
    ii = jnp.arange(e, dtype=jnp.int32)
    cum = jnp.sum(jnp.where(ii[None, :] <= ii[:, None], ntiles[None, :], 0), axis=1)
    total = jnp.sum(ntiles)
    u = jnp.arange(n_units, dtype=jnp.int32)
    eu = jnp.sum((cum[None, :] <= u[:, None]).astype(jnp.int32), axis=1)
    euc = jnp.minimum(eu, e - 1)
    oh = ii[None, :] == euc[:, None]

    def sel(v):
        return jnp.sum(jnp.where(oh, v[None, :], 0), axis=1)

    prev = sel(cum) - sel(ntiles)
    tile_u = sel(first_tile) + (u - prev)
    rs_u = jnp.maximum(sel(lo), tile_u * tt)
    re_u = jnp.minimum(sel(hi), (tile_u + 1) * tt)
    valid = u < total
    last = jnp.maximum(total - 1, 0)
    ohl = (u == last) & (total > 0)
    e_pad = jnp.sum(jnp.where(ohl, euc, 0))
    t_pad = jnp.sum(jnp.where(ohl, tile_u, 0))
    return jnp.stack([
        jnp.where(valid, euc, e_pad),
        jnp.where(valid, tile_u, t_pad),
        jnp.where(valid, rs_u, 0),
        jnp.where(valid, re_u, 0),
    ])


@jax.jit
def kernel(hidden_states, experts_indices, start_indices, end_indices, gate_w, up_w, down_w):
    del experts_indices
    t_tokens, d = hidden_states.shape
    e, ff, _ = gate_w.shape
    tt = _TT
    n_tiles = t_tokens // tt
    n_units = n_tiles + e

    s = start_indices.astype(jnp.int32)
    seg_lo = s
    seg_hi = jnp.minimum(
        end_indices.astype(jnp.int32),
        jnp.concatenate([s[1:], jnp.full((1,), t_tokens, jnp.int32)]),
    )
    meta = _build_units(seg_lo, seg_hi, n_tiles, tt, n_units)

    grid_spec = pltpu.PrefetchScalarGridSpec(
        num_scalar_prefetch=1,
        grid=(n_units,),
        in_specs=[
            pl.BlockSpec((tt, d), lambda g, m: (m[1, g], 0)),
            pl.BlockSpec((1, ff, d), lambda g, m: (m[0, g], 0, 0)),
            pl.BlockSpec((1, ff, d), lambda g, m: (m[0, g], 0, 0)),
            pl.BlockSpec((1, d, ff), lambda g, m: (m[0, g], 0, 0)),
        ],
        out_specs=pl.BlockSpec((t_tokens, d), lambda g, m: (0, 0)),
    )
    return pl.pallas_call(
        functools.partial(_moe_unit_kernel, tt=tt),
        grid_spec=grid_spec,
        out_shape=jax.ShapeDtypeStruct((t_tokens, d), jnp.float32),
        compiler_params=pltpu.CompilerParams(
            dimension_semantics=("arbitrary",),
        ),
    )(meta, hidden_states, gate_w, up_w, down_w)
